# baseline (speedup 1.0000x reference)
import jax
import jax.numpy as jnp
from jax.experimental import pallas as pl
from jax.experimental.pallas import tpu as pltpu

_R = 256


def _issue_block(idx_ref, conf_hbm, buf, sem, step):
    base = step * _R

    def issue_one(k, carry):
        row = idx_ref[base + k]
        grp = (row // 8) * 8
        pltpu.make_async_copy(
            conf_hbm.at[pl.ds(grp, 8), :],
            buf.at[k],
            sem,
        ).start()
        return carry

    jax.lax.fori_loop(0, _R, issue_one, 0, unroll=8)


def _wait_block(conf_hbm, buf, sem):
    def wait_one(k, carry):
        pltpu.make_async_copy(
            conf_hbm.at[pl.ds(0, 8), :],
            buf.at[0],
            sem,
        ).wait()
        return carry

    jax.lax.fori_loop(0, _R, wait_one, 0, unroll=8)


def _body(idx_ref, x_ref, smod_ref, conf_hbm, loss_ref, acc_ref, buf, sem):
    i = pl.program_id(0)
    nsteps = pl.num_programs(0)
    par = jax.lax.rem(i, 2)
    nxt = jax.lax.rem(i + 1, 2)

    @pl.when(i == 0)
    def _():
        _issue_block(idx_ref, conf_hbm, buf.at[0], sem.at[0], 0)

    @pl.when(i + 1 < nsteps)
    def _():
        _issue_block(idx_ref, conf_hbm, buf.at[nxt], sem.at[nxt], i + 1)

    _wait_block(conf_hbm, buf.at[par], sem.at[par])

    x = x_ref[...]
    sv = smod_ref[0]
    g = jnp.zeros_like(x)
    for j in range(8):
        g = g + jnp.where(sv == float(j), buf[par, :, j, :], 0.0)

    m = jnp.max(x, axis=1, keepdims=True)
    lse = m + jnp.log(jnp.sum(jnp.exp(x - m), axis=1, keepdims=True))
    s1 = jnp.sum(g, axis=1, keepdims=True)
    d = jnp.sum(x * g, axis=1, keepdims=True)
    loss = lse * s1 - d
    loss_ref[...] = loss

    @pl.when(i == 0)
    def _():
        acc_ref[...] = jnp.zeros_like(acc_ref)

    total = acc_ref[...] + jnp.sum(loss).reshape(1, 1)
    acc_ref[...] = total

    @pl.when(i == nsteps - 1)
    def _():
        acc_ref[...] = total / (nsteps * _R)


def kernel(outputs, index, confidence):
    B, C = outputs.shape
    G = B // _R
    smod = (index % 8).astype(jnp.float32).reshape(G, _R, 1)
    grid_spec = pltpu.PrefetchScalarGridSpec(
        num_scalar_prefetch=1,
        grid=(G,),
        in_specs=[
            pl.BlockSpec((_R, C), lambda i, idx: (i, 0)),
            pl.BlockSpec((1, _R, 1), lambda i, idx: (i, 0, 0)),
            pl.BlockSpec(memory_space=pl.ANY),
        ],
        out_specs=[
            pl.BlockSpec((_R, 1), lambda i, idx: (i, 0)),
            pl.BlockSpec((1, 1), lambda i, idx: (0, 0)),
        ],
        scratch_shapes=[
            pltpu.VMEM((2, _R, 8, C), jnp.float32),
            pltpu.SemaphoreType.DMA((2,)),
        ],
    )
    loss2, acc = pl.pallas_call(
        _body,
        grid_spec=grid_spec,
        out_shape=[
            jax.ShapeDtypeStruct((B, 1), jnp.float32),
            jax.ShapeDtypeStruct((1, 1), jnp.float32),
        ],
    )(index, outputs, smod, confidence)
    return (acc[0, 0], loss2.reshape(B))


# speedup vs baseline: 1.1275x; 1.1275x over previous
import jax
import jax.numpy as jnp
from jax.experimental import pallas as pl
from jax.experimental.pallas import tpu as pltpu

_R = 256
_Q = 8


def _issue_block(idx_ref, conf_hbm, buf, sems, step):
    base = step * _R

    def issue_one(k, carry):
        row = idx_ref[base + k]
        pltpu.make_async_copy(
            conf_hbm.at[pl.ds(row, 1), :],
            buf.at[pl.ds(k, 1), :],
            sems.at[jax.lax.rem(k, _Q)],
        ).start()
        return carry

    jax.lax.fori_loop(0, _R, issue_one, 0, unroll=_Q)


def _wait_block(conf_hbm, buf, sems):
    def wait_one(k, carry):
        pltpu.make_async_copy(
            conf_hbm.at[pl.ds(0, 1), :],
            buf.at[pl.ds(0, 1), :],
            sems.at[jax.lax.rem(k, _Q)],
        ).wait()
        return carry

    jax.lax.fori_loop(0, _R, wait_one, 0, unroll=_Q)


def _body(idx_ref, x_ref, conf_hbm, loss_ref, acc_ref, buf, sems0, sems1):
    i = pl.program_id(0)
    nsteps = pl.num_programs(0)
    par = jax.lax.rem(i, 2)
    nxt = jax.lax.rem(i + 1, 2)

    @pl.when(i == 0)
    def _():
        _issue_block(idx_ref, conf_hbm, buf.at[0], sems0, 0)

    @pl.when((i + 1 < nsteps) & (nxt == 1))
    def _():
        _issue_block(idx_ref, conf_hbm, buf.at[1], sems1, i + 1)

    @pl.when((i + 1 < nsteps) & (nxt == 0))
    def _():
        _issue_block(idx_ref, conf_hbm, buf.at[0], sems0, i + 1)

    @pl.when(par == 0)
    def _():
        _wait_block(conf_hbm, buf.at[0], sems0)

    @pl.when(par == 1)
    def _():
        _wait_block(conf_hbm, buf.at[1], sems1)

    x = x_ref[...]
    g = buf[par]
    m = jnp.max(x, axis=1, keepdims=True)
    lse = m + jnp.log(jnp.sum(jnp.exp(x - m), axis=1, keepdims=True))
    s1 = jnp.sum(g, axis=1, keepdims=True)
    d = jnp.sum(x * g, axis=1, keepdims=True)
    loss = lse * s1 - d
    loss_ref[...] = loss

    @pl.when(i == 0)
    def _():
        acc_ref[...] = jnp.zeros_like(acc_ref)

    total = acc_ref[...] + jnp.sum(loss).reshape(1, 1)
    acc_ref[...] = total

    @pl.when(i == nsteps - 1)
    def _():
        acc_ref[...] = total / (nsteps * _R)


def kernel(outputs, index, confidence):
    B, C = outputs.shape
    G = B // _R
    grid_spec = pltpu.PrefetchScalarGridSpec(
        num_scalar_prefetch=1,
        grid=(G,),
        in_specs=[
            pl.BlockSpec((_R, C), lambda i, idx: (i, 0)),
            pl.BlockSpec(memory_space=pl.ANY),
        ],
        out_specs=[
            pl.BlockSpec((_R, 1), lambda i, idx: (i, 0)),
            pl.BlockSpec((1, 1), lambda i, idx: (0, 0)),
        ],
        scratch_shapes=[
            pltpu.VMEM((2, _R, C), jnp.float32),
            pltpu.SemaphoreType.DMA((_Q,)),
            pltpu.SemaphoreType.DMA((_Q,)),
        ],
    )
    loss2, acc = pl.pallas_call(
        _body,
        grid_spec=grid_spec,
        out_shape=[
            jax.ShapeDtypeStruct((B, 1), jnp.float32),
            jax.ShapeDtypeStruct((1, 1), jnp.float32),
        ],
    )(index, outputs, confidence)
    return (acc[0, 0], loss2.reshape(B))


# speedup vs baseline: 1.2701x; 1.1265x over previous
import jax
import jax.numpy as jnp
from jax.experimental import pallas as pl
from jax.experimental.pallas import tpu as pltpu

_R = 512


def _issue_block(idx_ref, conf_hbm, buf, sem, step):
    base = step * _R

    def issue_one(k, carry):
        row = idx_ref[base + k]
        pltpu.make_async_copy(
            conf_hbm.at[pl.ds(row, 1), :],
            buf.at[pl.ds(k, 1), :],
            sem,
        ).start()
        return carry

    jax.lax.fori_loop(0, _R, issue_one, 0, unroll=8)


def _wait_block(conf_hbm, buf, sem):
    pltpu.make_async_copy(conf_hbm.at[pl.ds(0, _R), :], buf, sem).wait()


def _body(idx_ref, x_ref, conf_hbm, loss_ref, acc_ref, buf, sem):
    i = pl.program_id(0)
    nsteps = pl.num_programs(0)
    par = jax.lax.rem(i, 2)
    nxt = jax.lax.rem(i + 1, 2)

    @pl.when(i == 0)
    def _():
        _issue_block(idx_ref, conf_hbm, buf.at[0], sem.at[0], 0)

    @pl.when(i + 1 < nsteps)
    def _():
        _issue_block(idx_ref, conf_hbm, buf.at[nxt], sem.at[nxt], i + 1)

    _wait_block(conf_hbm, buf.at[par], sem.at[par])

    x = x_ref[...]
    g = buf[par]
    m = jnp.max(x, axis=1, keepdims=True)
    lse = m + jnp.log(jnp.sum(jnp.exp(x - m), axis=1, keepdims=True))
    s1 = jnp.sum(g, axis=1, keepdims=True)
    d = jnp.sum(x * g, axis=1, keepdims=True)
    loss = lse * s1 - d
    loss_ref[...] = loss

    @pl.when(i == 0)
    def _():
        acc_ref[...] = jnp.zeros_like(acc_ref)

    total = acc_ref[...] + jnp.sum(loss).reshape(1, 1)
    acc_ref[...] = total

    @pl.when(i == nsteps - 1)
    def _():
        acc_ref[...] = total / (nsteps * _R)


def kernel(outputs, index, confidence):
    B, C = outputs.shape
    G = B // _R
    grid_spec = pltpu.PrefetchScalarGridSpec(
        num_scalar_prefetch=1,
        grid=(G,),
        in_specs=[
            pl.BlockSpec((_R, C), lambda i, idx: (i, 0)),
            pl.BlockSpec(memory_space=pl.ANY),
        ],
        out_specs=[
            pl.BlockSpec((_R, 1), lambda i, idx: (i, 0)),
            pl.BlockSpec((1, 1), lambda i, idx: (0, 0)),
        ],
        scratch_shapes=[
            pltpu.VMEM((2, _R, C), jnp.float32),
            pltpu.SemaphoreType.DMA((2,)),
        ],
    )
    loss2, acc = pl.pallas_call(
        _body,
        grid_spec=grid_spec,
        out_shape=[
            jax.ShapeDtypeStruct((B, 1), jnp.float32),
            jax.ShapeDtypeStruct((1, 1), jnp.float32),
        ],
    )(index, outputs, confidence)
    return (acc[0, 0], loss2.reshape(B))
